# baseline (speedup 1.0000x reference)
import functools

import jax
import jax.numpy as jnp
from jax import lax
from jax.experimental import pallas as pl
from jax.experimental.pallas import tpu as pltpu
from jax.experimental.pallas import tpu_sc as plsc

_K = 8192
_D = 128
_N = 4608
_BETA = 0.25
_TEMP = 0.1
_NB = 512
_NW = 32
_BPW = _N // _NW


def _vq_body(z_ref, znrm_ref, w_ref, wnrm_ref,
             idx_ref, ent_ref, wn_ref, psum_ref, n_rows, rb):
    r = pl.program_id(0)

    @pl.when(r == 0)
    def _init():
        wn_ref[...] = w_ref[...] / wnrm_ref[...]
        psum_ref[...] = jnp.zeros_like(psum_ref)

    zn = z_ref[...] / znrm_ref[...]
    c = jax.lax.dot_general(zn, wn_ref[...], (((1,), (1,)), ((), ())),
                            preferred_element_type=jnp.float32)
    idx_ref[...] = jnp.argmax(c, axis=1).astype(jnp.int32).reshape(-1, 1)
    e = jnp.exp(c * (1.0 / _TEMP)).astype(jnp.bfloat16)
    ones_k = jnp.ones((_K, 1), jnp.bfloat16)
    s = jax.lax.dot_general(e, ones_k, (((1,), (0,)), ((), ())),
                            preferred_element_type=jnp.float32)
    psum_ref[...] += jax.lax.dot_general(
        (1.0 / s).astype(jnp.bfloat16), e, (((0,), (0,)), ((), ())),
        preferred_element_type=jnp.float32)

    @pl.when(r == rb - 1)
    def _finalize():
        pavg = psum_ref[...] / n_rows + 1e-8
        ent_ref[...] = -jnp.sum(pavg * jnp.log(pavg)).reshape(1, 1)


def _sc_body(w_hbm, idx_hbm, zeros_hbm, ones_hbm, zq_hbm, cnt_hbm,
             idx_v, rows_v, ones_v, cnt_v, cnt_sh, sem):
    cid = lax.axis_index("c")
    sid = lax.axis_index("s")
    wid = cid * 16 + sid
    base = wid * _BPW

    @pl.when(sid == 0)
    def _zero():
        pltpu.sync_copy(zeros_hbm, cnt_v)
        pltpu.sync_copy(cnt_v, cnt_sh)

    plsc.subcore_barrier()
    pltpu.sync_copy(idx_hbm.at[pl.ds(base, _BPW)], idx_v)
    pltpu.async_copy(w_hbm.at[idx_v], rows_v, sem).wait()
    pltpu.sync_copy(rows_v, zq_hbm.at[pl.ds(base, _BPW)])
    pltpu.sync_copy(ones_hbm, ones_v)
    pltpu.sync_copy(ones_v, cnt_sh.at[idx_v], add=True)
    plsc.subcore_barrier()

    @pl.when(sid == 0)
    def _out():
        pltpu.sync_copy(cnt_sh, cnt_v)
        pltpu.sync_copy(cnt_v, cnt_hbm.at[cid])


def _epi_body(z_ref, zq_ref, cnt_ref, com_ref, ppl_ref):
    diff = zq_ref[...] - z_ref[...]
    com_ref[...] = ((1.0 + _BETA) * jnp.sum(diff * diff)
                    / (_N * _D)).reshape(1, 1)
    e_mean = (cnt_ref[0:1, :] + cnt_ref[1:2, :]) / _N
    ppl_ref[...] = jnp.exp(
        -jnp.sum(e_mean * jnp.log(e_mean + 1e-8))).reshape(1, 1)


@jax.jit
def _cos_vq(z_flat, znrm, W, wnrm):
    n = z_flat.shape[0]
    rb = n // _NB
    idx, ent = pl.pallas_call(
        functools.partial(_vq_body, n_rows=n, rb=rb),
        grid=(rb,),
        in_specs=[
            pl.BlockSpec((_NB, _D), lambda r: (r, 0)),
            pl.BlockSpec((_NB, 1), lambda r: (r, 0)),
            pl.BlockSpec((_K, _D), lambda r: (0, 0)),
            pl.BlockSpec((_K, 1), lambda r: (0, 0)),
        ],
        out_specs=[
            pl.BlockSpec((_NB, 1), lambda r: (r, 0)),
            pl.BlockSpec((1, 1), lambda r: (0, 0)),
        ],
        out_shape=[
            jax.ShapeDtypeStruct((n, 1), jnp.int32),
            jax.ShapeDtypeStruct((1, 1), jnp.float32),
        ],
        scratch_shapes=[
            pltpu.VMEM((_K, _D), jnp.float32),
            pltpu.VMEM((1, _K), jnp.float32),
        ],
    )(z_flat, znrm, W, wnrm)

    mesh = plsc.VectorSubcoreMesh(core_axis_name="c", subcore_axis_name="s")
    zq, cnt2 = functools.partial(
        pl.kernel, mesh=mesh,
        out_type=[
            jax.ShapeDtypeStruct((n, _D), jnp.float32),
            jax.ShapeDtypeStruct((2, _K), jnp.float32),
        ],
        scratch_types=[
            pltpu.VMEM((_BPW,), jnp.int32),
            pltpu.VMEM((_BPW, _D), jnp.float32),
            pltpu.VMEM((_BPW,), jnp.float32),
            pltpu.VMEM((_K,), jnp.float32),
            pltpu.VMEM_SHARED((_K,), jnp.float32),
            pltpu.SemaphoreType.DMA,
        ],
    )(_sc_body)(W, idx.reshape(-1),
                jnp.zeros((_K,), jnp.float32),
                jnp.ones((_BPW,), jnp.float32))

    com, ppl = pl.pallas_call(
        _epi_body,
        grid=(1,),
        in_specs=[
            pl.BlockSpec((n, _D), lambda r: (0, 0)),
            pl.BlockSpec((n, _D), lambda r: (0, 0)),
            pl.BlockSpec((2, _K), lambda r: (0, 0)),
        ],
        out_specs=[
            pl.BlockSpec((1, 1), lambda r: (0, 0)),
            pl.BlockSpec((1, 1), lambda r: (0, 0)),
        ],
        out_shape=[
            jax.ShapeDtypeStruct((1, 1), jnp.float32),
            jax.ShapeDtypeStruct((1, 1), jnp.float32),
        ],
    )(z_flat, zq, cnt2)

    return zq, com[0, 0], ppl[0, 0], ent[0, 0]


def kernel(z, W):
    z_flat = z.reshape(-1, _D)
    znrm = jnp.maximum(jnp.linalg.norm(z_flat, axis=1, keepdims=True), 1e-12)
    wnrm = jnp.maximum(jnp.linalg.norm(W, axis=1, keepdims=True), 1e-12)
    zq, com, ppl, ent = _cos_vq(z_flat, znrm, W, wnrm)
    return zq.reshape(z.shape), com, ppl, ent


# speedup vs baseline: 1.0229x; 1.0229x over previous
import functools

import jax
import jax.numpy as jnp
from jax import lax
from jax.experimental import pallas as pl
from jax.experimental.pallas import tpu as pltpu
from jax.experimental.pallas import tpu_sc as plsc

_K = 8192
_D = 128
_N = 4608
_BETA = 0.25
_TEMP = 0.1
_NB = 512
_NW = 32
_BPW = _N // _NW


def _vq_body(z_ref, znrm_ref, w_ref, wnrm_ref,
             idx_ref, ent_ref, wn_ref, psum_ref, n_rows, rb):
    r = pl.program_id(0)

    @pl.when(r == 0)
    def _init():
        wn_ref[...] = w_ref[...] / wnrm_ref[...]
        psum_ref[...] = jnp.zeros_like(psum_ref)

    zn = z_ref[...] / znrm_ref[...]
    c = jax.lax.dot_general(zn, wn_ref[...], (((1,), (1,)), ((), ())),
                            preferred_element_type=jnp.float32)
    m = jnp.max(c, axis=1, keepdims=True)
    colidx = jax.lax.broadcasted_iota(jnp.int32, c.shape, 1)
    idx_ref[...] = jnp.min(jnp.where(c == m, colidx, _K),
                           axis=1, keepdims=True)
    e = jnp.exp(c * (1.0 / _TEMP)).astype(jnp.bfloat16)
    ones_k = jnp.ones((_K, 1), jnp.bfloat16)
    s = jax.lax.dot_general(e, ones_k, (((1,), (0,)), ((), ())),
                            preferred_element_type=jnp.float32)
    psum_ref[...] += jax.lax.dot_general(
        (1.0 / s).astype(jnp.bfloat16), e, (((0,), (0,)), ((), ())),
        preferred_element_type=jnp.float32)

    @pl.when(r == rb - 1)
    def _finalize():
        pavg = psum_ref[...] / n_rows + 1e-8
        ent_ref[...] = -jnp.sum(pavg * jnp.log(pavg)).reshape(1, 1)


def _sc_body(w_hbm, idx_hbm, zeros_hbm, ones_hbm, zq_hbm, cnt_hbm,
             idx_v, rows_v, ones_v, cnt_v, cnt_sh, sem):
    cid = lax.axis_index("c")
    sid = lax.axis_index("s")
    wid = cid * 16 + sid
    base = wid * _BPW

    @pl.when(sid == 0)
    def _zero():
        pltpu.sync_copy(zeros_hbm, cnt_v)
        pltpu.sync_copy(cnt_v, cnt_sh)

    plsc.subcore_barrier()
    pltpu.sync_copy(idx_hbm.at[pl.ds(base, _BPW)], idx_v)
    pltpu.async_copy(w_hbm.at[idx_v], rows_v, sem).wait()
    pltpu.sync_copy(rows_v, zq_hbm.at[pl.ds(base, _BPW)])
    pltpu.sync_copy(ones_hbm, ones_v)
    pltpu.sync_copy(ones_v, cnt_sh.at[idx_v], add=True)
    plsc.subcore_barrier()

    @pl.when(sid == 0)
    def _out():
        pltpu.sync_copy(cnt_sh, cnt_v)
        pltpu.sync_copy(cnt_v, cnt_hbm.at[cid])


def _epi_body(z_ref, zq_ref, cnt_ref, com_ref, ppl_ref):
    diff = zq_ref[...] - z_ref[...]
    com_ref[...] = ((1.0 + _BETA) * jnp.sum(diff * diff)
                    / (_N * _D)).reshape(1, 1)
    e_mean = (cnt_ref[0:1, :] + cnt_ref[1:2, :]) / _N
    ppl_ref[...] = jnp.exp(
        -jnp.sum(e_mean * jnp.log(e_mean + 1e-8))).reshape(1, 1)


@jax.jit
def _cos_vq(z_flat, znrm, W, wnrm):
    n = z_flat.shape[0]
    rb = n // _NB
    idx, ent = pl.pallas_call(
        functools.partial(_vq_body, n_rows=n, rb=rb),
        grid=(rb,),
        in_specs=[
            pl.BlockSpec((_NB, _D), lambda r: (r, 0)),
            pl.BlockSpec((_NB, 1), lambda r: (r, 0)),
            pl.BlockSpec((_K, _D), lambda r: (0, 0)),
            pl.BlockSpec((_K, 1), lambda r: (0, 0)),
        ],
        out_specs=[
            pl.BlockSpec((_NB, 1), lambda r: (r, 0)),
            pl.BlockSpec((1, 1), lambda r: (0, 0)),
        ],
        out_shape=[
            jax.ShapeDtypeStruct((n, 1), jnp.int32),
            jax.ShapeDtypeStruct((1, 1), jnp.float32),
        ],
        scratch_shapes=[
            pltpu.VMEM((_K, _D), jnp.float32),
            pltpu.VMEM((1, _K), jnp.float32),
        ],
    )(z_flat, znrm, W, wnrm)

    mesh = plsc.VectorSubcoreMesh(core_axis_name="c", subcore_axis_name="s")
    zq, cnt2 = functools.partial(
        pl.kernel, mesh=mesh,
        out_type=[
            jax.ShapeDtypeStruct((n, _D), jnp.float32),
            jax.ShapeDtypeStruct((2, _K), jnp.float32),
        ],
        scratch_types=[
            pltpu.VMEM((_BPW,), jnp.int32),
            pltpu.VMEM((_BPW, _D), jnp.float32),
            pltpu.VMEM((_BPW,), jnp.float32),
            pltpu.VMEM((_K,), jnp.float32),
            pltpu.VMEM_SHARED((_K,), jnp.float32),
            pltpu.SemaphoreType.DMA,
        ],
    )(_sc_body)(W, idx.reshape(-1),
                jnp.zeros((_K,), jnp.float32),
                jnp.ones((_BPW,), jnp.float32))

    com, ppl = pl.pallas_call(
        _epi_body,
        grid=(1,),
        in_specs=[
            pl.BlockSpec((n, _D), lambda r: (0, 0)),
            pl.BlockSpec((n, _D), lambda r: (0, 0)),
            pl.BlockSpec((2, _K), lambda r: (0, 0)),
        ],
        out_specs=[
            pl.BlockSpec((1, 1), lambda r: (0, 0)),
            pl.BlockSpec((1, 1), lambda r: (0, 0)),
        ],
        out_shape=[
            jax.ShapeDtypeStruct((1, 1), jnp.float32),
            jax.ShapeDtypeStruct((1, 1), jnp.float32),
        ],
    )(z_flat, zq, cnt2)

    return zq, com[0, 0], ppl[0, 0], ent[0, 0]


def kernel(z, W):
    z_flat = z.reshape(-1, _D)
    znrm = jnp.maximum(jnp.linalg.norm(z_flat, axis=1, keepdims=True), 1e-12)
    wnrm = jnp.maximum(jnp.linalg.norm(W, axis=1, keepdims=True), 1e-12)
    zq, com, ppl, ent = _cos_vq(z_flat, znrm, W, wnrm)
    return zq.reshape(z.shape), com, ppl, ent
